# baseline (speedup 1.0000x reference)
import functools

import jax
import jax.numpy as jnp
from jax import lax
from jax.experimental import pallas as pl
from jax.experimental.pallas import tpu as pltpu
from jax.experimental.pallas import tpu_sc as plsc

PLACEHOLDER = -1
VOCAB = 100000
SPEC = 4
BATCH = 32
NROW = BATCH * SPEC
L = 16
INT_MAX = 2**31 - 1

V_TC = 76800
TC_B0 = 2560
TC_NG = V_TC // TC_B0
SC_ROWS = 24576
S0 = VOCAB - SC_ROWS
W_ROWS = SC_ROWS // 32
CROWS = 128
SC_NCH = W_ROWS // CROWS


def _sc_body(t_hbm, d_hbm, q_hbm, ids_hbm,
             oval, oidx, opp,
             tb0, tb1, db0, db1, qb0, qb1,
             ids_v, tmpf_v, obuf_v, obi_v, idxg_v, gt_v, gd_v,
             sem0, sem1, gsem):
    c = lax.axis_index("c")
    s = lax.axis_index("s")
    w = c * 16 + s
    rs = S0 + w * W_ROWS
    iota = lax.iota(jnp.int32, L)

    pltpu.sync_copy(ids_hbm, ids_v)

    tbufs = (tb0, tb1)
    dbufs = (db0, db1)
    qbufs = (qb0, qb1)
    sems = (sem0, sem1)

    def copies(ci, k):
        r0 = pl.multiple_of(rs + ci * CROWS, 8)
        return (
            pltpu.make_async_copy(
                t_hbm.at[pl.ds(r0, CROWS)], tbufs[k], sems[k]),
            pltpu.make_async_copy(
                d_hbm.at[pl.ds(r0, CROWS)], dbufs[k], sems[k]),
            pltpu.make_async_copy(
                q_hbm.at[pl.ds(r0, CROWS)], qbufs[k], sems[k]),
        )

    def issue(ci, k):
        for cp in copies(ci, k):
            cp.start()

    def drain(ci, k):
        for cp in copies(ci, k):
            cp.wait()

    accs = []
    for _ in range(8):
        accs += [jnp.full((L,), -1.0, jnp.float32),
                 jnp.ones((L,), jnp.float32),
                 jnp.zeros((L,), jnp.int32)]

    issue(0, 0)
    issue(1, 1)

    def do_chunk(ci, k, carry):
        drain(ci, k)
        tb, db, qb = tbufs[k], dbufs[k], qbufs[k]
        base = rs + ci * CROWS

        def tbody(kt, pc):
            pc = list(pc)
            for r in range(8):
                row = kt * 8 + r
                vrow = jnp.full((L,), base + row, jnp.int32)
                for p in range(8):
                    t = tb[row, pl.ds(p * L, L)]
                    d = db[row, pl.ds(p * L, L)]
                    qv = qb[row, pl.ds(p * L, L)]
                    diff = jnp.maximum(t - d, 0.0)
                    bd, bq, bi = pc[3 * p], pc[3 * p + 1], pc[3 * p + 2]
                    better = diff * bq > bd * qv
                    pc[3 * p] = jnp.where(better, diff, bd)
                    pc[3 * p + 1] = jnp.where(better, qv, bq)
                    pc[3 * p + 2] = jnp.where(better, vrow, bi)
            return tuple(pc)

        return list(lax.fori_loop(0, CROWS // 8, tbody, tuple(carry)))

    def two_chunks(i, carry):
        carry = do_chunk(2 * i, 0, carry)

        @pl.when(2 * i + 2 < SC_NCH)
        def _():
            issue(2 * i + 2, 0)

        carry = do_chunk(2 * i + 1, 1, carry)

        @pl.when(2 * i + 3 < SC_NCH)
        def _():
            issue(2 * i + 3, 1)
        return tuple(carry)

    accs = list(lax.fori_loop(0, SC_NCH // 2, two_chunks, tuple(accs)))

    for p in range(8):
        bd, bq, bi = accs[3 * p], accs[3 * p + 1], accs[3 * p + 2]
        obuf_v[pl.ds(p * L, L)] = bd / bq
        obi_v[pl.ds(p * L, L)] = bi
    pltpu.sync_copy(obuf_v, oval.at[pl.ds(w * NROW, NROW)])
    pltpu.sync_copy(obi_v, oidx.at[pl.ds(w * NROW, NROW)])

    k4 = jnp.minimum(iota & 7, SPEC - 1)
    toks = w * SPEC + k4
    idxg_v[...] = plsc.load_gather(ids_v, [toks])
    pltpu.async_copy(t_hbm.at[idxg_v], gt_v, gsem).wait()
    pltpu.async_copy(d_hbm.at[idxg_v], gd_v, gsem).wait()
    gd = plsc.load_gather(gd_v, [iota, toks])
    gt = plsc.load_gather(gt_v, [iota, toks])
    pp = jnp.where(iota < 8, gd, gt)
    tmpf_v[...] = pp
    pltpu.sync_copy(tmpf_v, opp.at[pl.ds(w * L, L)])


@functools.lru_cache(maxsize=1)
def _build_sc():
    mesh = plsc.VectorSubcoreMesh(core_axis_name="c", subcore_axis_name="s")
    return pl.kernel(
        _sc_body,
        out_type=(
            jax.ShapeDtypeStruct((32 * NROW,), jnp.float32),
            jax.ShapeDtypeStruct((32 * NROW,), jnp.int32),
            jax.ShapeDtypeStruct((32 * L,), jnp.float32),
        ),
        mesh=mesh,
        compiler_params=pltpu.CompilerParams(needs_layout_passes=False),
        scratch_types=[
            pltpu.VMEM((CROWS, 128), jnp.float32),
            pltpu.VMEM((CROWS, 128), jnp.float32),
            pltpu.VMEM((CROWS, 128), jnp.float32),
            pltpu.VMEM((CROWS, 128), jnp.float32),
            pltpu.VMEM((CROWS, 128), jnp.float32),
            pltpu.VMEM((CROWS, 128), jnp.float32),
            pltpu.VMEM((NROW,), jnp.int32),
            pltpu.VMEM((L,), jnp.float32),
            pltpu.VMEM((NROW,), jnp.float32),
            pltpu.VMEM((NROW,), jnp.int32),
            pltpu.VMEM((L,), jnp.int32),
            pltpu.VMEM((L, 128), jnp.float32),
            pltpu.VMEM((L, 128), jnp.float32),
            pltpu.SemaphoreType.DMA,
            pltpu.SemaphoreType.DMA,
            pltpu.SemaphoreType.DMA,
        ],
    )


def _tc_scan_body(t_ref, d_ref, q_ref, m_ref, i_ref):
    g = pl.program_id(0)
    val = jnp.maximum(t_ref[...] - d_ref[...], 0.0) / q_ref[...]
    first = g == 0
    m_prev = jnp.where(first, -1.0, m_ref[...])
    i_prev = jnp.where(first, 0, i_ref[...])
    better = val > m_prev
    m_ref[...] = jnp.where(better, val, m_prev)
    i_ref[...] = jnp.where(better, g, i_prev)


def _tc_scan(t, d, q):
    blk = pl.BlockSpec((TC_B0, NROW), lambda i: (i, 0))
    st = pl.BlockSpec((TC_B0, NROW), lambda i: (0, 0))
    return pl.pallas_call(
        _tc_scan_body,
        grid=(TC_NG,),
        in_specs=[blk, blk, blk],
        out_specs=[st, st],
        out_shape=[
            jax.ShapeDtypeStruct((TC_B0, NROW), jnp.float32),
            jax.ShapeDtypeStruct((TC_B0, NROW), jnp.int32),
        ],
        compiler_params=pltpu.CompilerParams(
            dimension_semantics=("arbitrary",)),
    )(t, d, q)


def _m1_body(tcm, tcg, scv, sci, rec_ref):
    mv = tcm[...]
    tm = jnp.max(mv, axis=0, keepdims=True)
    ri = lax.broadcasted_iota(jnp.int32, (TC_B0, NROW), 0)
    cand_i = tcg[...] * TC_B0 + ri
    ti = jnp.min(jnp.where(mv == tm, cand_i, INT_MAX), axis=0,
                 keepdims=True)
    vm = jnp.max(scv[...], axis=0, keepdims=True)
    im = jnp.min(jnp.where(scv[...] == vm, sci[...], INT_MAX), axis=0,
                 keepdims=True)
    best_tc = tm >= vm
    rec_ref[...] = jnp.where(best_tc, ti, im)


def _merge1(tcm, tcg, scv, sci):
    vec = pl.BlockSpec((1, NROW), lambda i: (0, 0))
    m32 = pl.BlockSpec((32, NROW), lambda i: (0, 0))
    st = pl.BlockSpec((TC_B0, NROW), lambda i: (0, 0))
    return pl.pallas_call(
        _m1_body,
        grid=(1,),
        in_specs=[st, st, m32, m32],
        out_specs=vec,
        out_shape=jax.ShapeDtypeStruct((1, NROW), jnp.int32),
    )(tcm, tcg, scv, sci)


def _m2_body(rec, dp, tp, ids, uni, bon, gre, out_ref):
    accm = (dp[...] > 0.0) & ((tp[...] / jnp.maximum(dp[...], 1e-30))
                              >= uni[...])
    token = jnp.where(accm, ids[...], rec[...])
    cols = []
    cum = jnp.ones((BATCH, 1), jnp.bool_)
    for k in range(SPEC):
        cols.append(jnp.where(cum, token[:, k:k + 1], PLACEHOLDER))
        cum = cum & accm[:, k:k + 1]
    cols.append(jnp.where(cum, bon[...], PLACEHOLDER))
    out = jnp.concatenate(cols, axis=1)
    out_ref[...] = jnp.where(gre[...] > 0, PLACEHOLDER, out)


def _merge2(*args):
    m4 = pl.BlockSpec((BATCH, SPEC), lambda i: (0, 0))
    m1 = pl.BlockSpec((BATCH, 1), lambda i: (0, 0))
    return pl.pallas_call(
        _m2_body,
        grid=(1,),
        in_specs=[m4, m4, m4, m4, m4, m1, m1],
        out_specs=pl.BlockSpec((BATCH, SPEC + 1), lambda i: (0, 0)),
        out_shape=jax.ShapeDtypeStruct((BATCH, SPEC + 1), jnp.int32),
    )(*args)


def kernel(draft_token_ids, draft_probs, target_probs, bonus_token_ids,
           uniform_probs, q, cu_num_draft_tokens, is_greedy):
    del cu_num_draft_tokens
    ids = draft_token_ids.astype(jnp.int32)
    bon = bonus_token_ids.reshape(-1).astype(jnp.int32)
    gre = is_greedy.astype(jnp.int32)

    tT = target_probs.T
    dT = draft_probs.T
    qT = q.T

    oval, oidx, opp = _build_sc()(tT, dT, qT, ids)
    scv = oval.reshape(32, NROW)
    sci = oidx.reshape(32, NROW)
    pp = opp.reshape(32, L)

    tcm, tci = _tc_scan(tT, dT, qT)
    rec = _merge1(tcm, tci, scv, sci)
    out = _merge2(rec.reshape(BATCH, SPEC), pp[:, :SPEC], pp[:, 8:8 + SPEC],
                  ids.reshape(BATCH, SPEC), uniform_probs.reshape(BATCH, SPEC),
                  bon.reshape(BATCH, 1), gre.reshape(BATCH, 1))
    return out.astype(draft_token_ids.dtype)


# speedup vs baseline: 1.0202x; 1.0202x over previous
import functools

import jax
import jax.numpy as jnp
from jax import lax
from jax.experimental import pallas as pl
from jax.experimental.pallas import tpu as pltpu
from jax.experimental.pallas import tpu_sc as plsc

PLACEHOLDER = -1
VOCAB = 100000
SPEC = 4
BATCH = 32
NROW = BATCH * SPEC
L = 16
INT_MAX = 2**31 - 1

V_TC = 76800
TC_B0 = 2560
TC_NG = V_TC // TC_B0
SC_ROWS = 24576
S0 = VOCAB - SC_ROWS
W_ROWS = SC_ROWS // 32
CROWS = 128
SC_NCH = W_ROWS // CROWS


def _sc_body(t_hbm, d_hbm, q_hbm, ids_hbm,
             oval, oidx, opp,
             tb0, tb1, db0, db1, qb0, qb1,
             ids_v, tmpf_v, obuf_v, obi_v, idxg_v, gt_v, gd_v,
             sem0, sem1, gsem):
    c = lax.axis_index("c")
    s = lax.axis_index("s")
    w = c * 16 + s
    rs = S0 + w * W_ROWS
    iota = lax.iota(jnp.int32, L)

    pltpu.sync_copy(ids_hbm, ids_v)

    tbufs = (tb0, tb1)
    dbufs = (db0, db1)
    qbufs = (qb0, qb1)
    sems = (sem0, sem1)

    def copies(ci, k):
        r0 = pl.multiple_of(rs + ci * CROWS, 8)
        return (
            pltpu.make_async_copy(
                t_hbm.at[pl.ds(r0, CROWS)], tbufs[k], sems[k]),
            pltpu.make_async_copy(
                d_hbm.at[pl.ds(r0, CROWS)], dbufs[k], sems[k]),
            pltpu.make_async_copy(
                q_hbm.at[pl.ds(r0, CROWS)], qbufs[k], sems[k]),
        )

    def issue(ci, k):
        for cp in copies(ci, k):
            cp.start()

    def drain(ci, k):
        for cp in copies(ci, k):
            cp.wait()

    accs = []
    for _ in range(8):
        accs += [jnp.full((L,), -1.0, jnp.float32),
                 jnp.ones((L,), jnp.float32),
                 jnp.zeros((L,), jnp.int32)]

    issue(0, 0)
    issue(1, 1)

    def do_chunk(ci, k, carry):
        drain(ci, k)
        tb, db, qb = tbufs[k], dbufs[k], qbufs[k]
        base = rs + ci * CROWS

        def tbody(kt, pc):
            pc = list(pc)
            for r in range(8):
                row = kt * 8 + r
                vrow = jnp.full((L,), base + row, jnp.int32)
                for p in range(8):
                    t = tb[row, pl.ds(p * L, L)]
                    d = db[row, pl.ds(p * L, L)]
                    qv = qb[row, pl.ds(p * L, L)]
                    diff = jnp.maximum(t - d, 0.0)
                    bd, bq, bi = pc[3 * p], pc[3 * p + 1], pc[3 * p + 2]
                    better = diff * bq > bd * qv
                    pc[3 * p] = jnp.where(better, diff, bd)
                    pc[3 * p + 1] = jnp.where(better, qv, bq)
                    pc[3 * p + 2] = jnp.where(better, vrow, bi)
            return tuple(pc)

        return list(lax.fori_loop(0, CROWS // 8, tbody, tuple(carry)))

    def two_chunks(i, carry):
        carry = do_chunk(2 * i, 0, carry)

        @pl.when(2 * i + 2 < SC_NCH)
        def _():
            issue(2 * i + 2, 0)

        carry = do_chunk(2 * i + 1, 1, carry)

        @pl.when(2 * i + 3 < SC_NCH)
        def _():
            issue(2 * i + 3, 1)
        return tuple(carry)

    accs = list(lax.fori_loop(0, SC_NCH // 2, two_chunks, tuple(accs)))

    for p in range(8):
        bd, bq, bi = accs[3 * p], accs[3 * p + 1], accs[3 * p + 2]
        obuf_v[pl.ds(p * L, L)] = bd / bq
        obi_v[pl.ds(p * L, L)] = bi
    pltpu.sync_copy(obuf_v, oval.at[pl.ds(w * NROW, NROW)])
    pltpu.sync_copy(obi_v, oidx.at[pl.ds(w * NROW, NROW)])

    k4 = jnp.minimum(iota & 7, SPEC - 1)
    toks = w * SPEC + k4
    idxg_v[...] = plsc.load_gather(ids_v, [toks])
    pltpu.async_copy(t_hbm.at[idxg_v], gt_v, gsem).wait()
    pltpu.async_copy(d_hbm.at[idxg_v], gd_v, gsem).wait()
    gd = plsc.load_gather(gd_v, [iota, toks])
    gt = plsc.load_gather(gt_v, [iota, toks])
    pp = jnp.where(iota < 8, gd, gt)
    tmpf_v[...] = pp
    pltpu.sync_copy(tmpf_v, opp.at[pl.ds(w * L, L)])


@functools.lru_cache(maxsize=1)
def _build_sc():
    mesh = plsc.VectorSubcoreMesh(core_axis_name="c", subcore_axis_name="s")
    return pl.kernel(
        _sc_body,
        out_type=(
            jax.ShapeDtypeStruct((32 * NROW,), jnp.float32),
            jax.ShapeDtypeStruct((32 * NROW,), jnp.int32),
            jax.ShapeDtypeStruct((32 * L,), jnp.float32),
        ),
        mesh=mesh,
        compiler_params=pltpu.CompilerParams(needs_layout_passes=False),
        scratch_types=[
            pltpu.VMEM((CROWS, 128), jnp.float32),
            pltpu.VMEM((CROWS, 128), jnp.float32),
            pltpu.VMEM((CROWS, 128), jnp.float32),
            pltpu.VMEM((CROWS, 128), jnp.float32),
            pltpu.VMEM((CROWS, 128), jnp.float32),
            pltpu.VMEM((CROWS, 128), jnp.float32),
            pltpu.VMEM((NROW,), jnp.int32),
            pltpu.VMEM((L,), jnp.float32),
            pltpu.VMEM((NROW,), jnp.float32),
            pltpu.VMEM((NROW,), jnp.int32),
            pltpu.VMEM((L,), jnp.int32),
            pltpu.VMEM((L, 128), jnp.float32),
            pltpu.VMEM((L, 128), jnp.float32),
            pltpu.SemaphoreType.DMA,
            pltpu.SemaphoreType.DMA,
            pltpu.SemaphoreType.DMA,
        ],
    )


def _tc_scan_body(t_ref, d_ref, q_ref, m_ref, i_ref):
    g = pl.program_id(0)
    base = g * TC_B0
    val = jnp.maximum(t_ref[...] - d_ref[...], 0.0) / q_ref[...]
    cm = jnp.max(val, axis=0, keepdims=True)
    ri = lax.broadcasted_iota(jnp.int32, (TC_B0, NROW), 0)
    ca = jnp.min(jnp.where(val == cm, ri, INT_MAX), axis=0,
                 keepdims=True) + base

    first = g == 0
    m_prev = jnp.where(first, -1.0, m_ref[...])
    i_prev = jnp.where(first, 0, i_ref[...])
    better = cm > m_prev
    m_ref[...] = jnp.where(better, cm, m_prev)
    i_ref[...] = jnp.where(better, ca, i_prev)


def _tc_scan(t, d, q):
    blk = pl.BlockSpec((TC_B0, NROW), lambda i: (i, 0))
    vec = pl.BlockSpec((1, NROW), lambda i: (0, 0))
    return pl.pallas_call(
        _tc_scan_body,
        grid=(TC_NG,),
        in_specs=[blk, blk, blk],
        out_specs=[vec, vec],
        out_shape=[
            jax.ShapeDtypeStruct((1, NROW), jnp.float32),
            jax.ShapeDtypeStruct((1, NROW), jnp.int32),
        ],
        compiler_params=pltpu.CompilerParams(
            dimension_semantics=("arbitrary",)),
    )(t, d, q)


def _m1_body(tcm, tci, scv, sci, rec_ref):
    vm = jnp.max(scv[...], axis=0, keepdims=True)
    im = jnp.min(jnp.where(scv[...] == vm, sci[...], INT_MAX), axis=0,
                 keepdims=True)
    best_tc = tcm[...] >= vm
    rec_ref[...] = jnp.where(best_tc, tci[...], im)


def _merge1(tcm, tci, scv, sci):
    vec = pl.BlockSpec((1, NROW), lambda i: (0, 0))
    m32 = pl.BlockSpec((32, NROW), lambda i: (0, 0))
    return pl.pallas_call(
        _m1_body,
        grid=(1,),
        in_specs=[vec, vec, m32, m32],
        out_specs=vec,
        out_shape=jax.ShapeDtypeStruct((1, NROW), jnp.int32),
    )(tcm, tci, scv, sci)


def _m2_body(rec, dp, tp, ids, uni, bon, gre, out_ref):
    accm = (dp[...] > 0.0) & ((tp[...] / jnp.maximum(dp[...], 1e-30))
                              >= uni[...])
    token = jnp.where(accm, ids[...], rec[...])
    cols = []
    cum = jnp.ones((BATCH, 1), jnp.bool_)
    for k in range(SPEC):
        cols.append(jnp.where(cum, token[:, k:k + 1], PLACEHOLDER))
        cum = cum & accm[:, k:k + 1]
    cols.append(jnp.where(cum, bon[...], PLACEHOLDER))
    out = jnp.concatenate(cols, axis=1)
    out_ref[...] = jnp.where(gre[...] > 0, PLACEHOLDER, out)


def _merge2(*args):
    m4 = pl.BlockSpec((BATCH, SPEC), lambda i: (0, 0))
    m1 = pl.BlockSpec((BATCH, 1), lambda i: (0, 0))
    return pl.pallas_call(
        _m2_body,
        grid=(1,),
        in_specs=[m4, m4, m4, m4, m4, m1, m1],
        out_specs=pl.BlockSpec((BATCH, SPEC + 1), lambda i: (0, 0)),
        out_shape=jax.ShapeDtypeStruct((BATCH, SPEC + 1), jnp.int32),
    )(*args)


def kernel(draft_token_ids, draft_probs, target_probs, bonus_token_ids,
           uniform_probs, q, cu_num_draft_tokens, is_greedy):
    del cu_num_draft_tokens
    ids = draft_token_ids.astype(jnp.int32)
    bon = bonus_token_ids.reshape(-1).astype(jnp.int32)
    gre = is_greedy.astype(jnp.int32)

    tT = target_probs.T
    dT = draft_probs.T
    qT = q.T

    oval, oidx, opp = _build_sc()(tT, dT, qT, ids)
    scv = oval.reshape(32, NROW)
    sci = oidx.reshape(32, NROW)
    pp = opp.reshape(32, L)

    tcm, tci = _tc_scan(tT, dT, qT)
    rec = _merge1(tcm, tci, scv, sci)
    out = _merge2(rec.reshape(BATCH, SPEC), pp[:, :SPEC], pp[:, 8:8 + SPEC],
                  ids.reshape(BATCH, SPEC), uniform_probs.reshape(BATCH, SPEC),
                  bon.reshape(BATCH, 1), gre.reshape(BATCH, 1))
    return out.astype(draft_token_ids.dtype)
